# baseline (speedup 1.0000x reference)
import jax
import jax.numpy as jnp
from jax import lax
from jax.experimental import pallas as pl
from jax.experimental.pallas import tpu as pltpu
from jax.experimental.pallas import tpu_sc as plsc

_B, _T, _H, _W = 1024, 8, 16, 16
_J = _T * _H * _W
_NE = 64
_VOCAB = 10
_NC, _NS, _L = 2, 16, 16
_NW = _NC * _NS
_SPW = _B // _NW
_BW = 2 * _L * _L


def _scatter_body(x_hbm, bins_hbm, col_hbm, xall, bins, cmat):
    cid = lax.axis_index("c")
    sid = lax.axis_index("s")
    wid = sid * _NC + cid
    base = wid * _SPW
    pltpu.sync_copy(x_hbm.at[pl.ds(base * _J, _SPW * _J)], xall)
    lane16 = lax.iota(jnp.int32, _L) * _L
    ones = jnp.ones((_L,), jnp.float32)
    zeros16 = jnp.zeros((_L,), jnp.float32)

    def zero_body(s, carry):
        for l in range(_BW // _L):
            bins[pl.ds(s * _BW + l * _L, _L)] = zeros16
        return carry
    lax.fori_loop(0, _SPW, zero_body, 0)

    def sample_body(s, carry):
        soff = s * _J
        vbase = lane16 + s * _BW
        rbase = vbase + _L * _L

        @plsc.parallel_loop(0, _J // _L, step=8, carry=zeros16)
        def colacc(i, acc):
            base8 = pl.multiple_of(soff + i * _L, _L * 8)
            for k in range(8):
                xv = xall[pl.ds(base8 + k * _L, _L)]
                maskf = jnp.minimum(xv, 1).astype(jnp.float32)
                plsc.addupdate_scatter(bins, [vbase + xv], ones)
                plsc.addupdate_scatter(bins, [rbase + (i + k) % _H], maskf)
                acc = acc + maskf
            return acc

        cmat[pl.ds(s * _L, _L)] = colacc
        return carry
    lax.fori_loop(0, _SPW, sample_body, 0)

    pltpu.sync_copy(bins, bins_hbm.at[pl.ds(base * _BW, _SPW * _BW)])
    pltpu.sync_copy(cmat, col_hbm.at[pl.ds(base * _L, _SPW * _L)])


def _sc_scatter(x2):
    mesh = plsc.VectorSubcoreMesh(core_axis_name="c", subcore_axis_name="s",
                                  num_cores=_NC, num_subcores=_NS)
    fn = pl.kernel(
        _scatter_body,
        out_type=(jax.ShapeDtypeStruct((_B * _BW,), jnp.float32),
                  jax.ShapeDtypeStruct((_B * _L,), jnp.float32)),
        mesh=mesh,
        compiler_params=pltpu.CompilerParams(needs_layout_passes=False),
        scratch_types=[
            pltpu.VMEM((_SPW * _J,), jnp.int32),
            pltpu.VMEM((_SPW * _BW,), jnp.float32),
            pltpu.VMEM((_SPW * _L,), jnp.float32),
        ],
    )
    return fn(x2)


def _combine_body(bins_ref, col_ref, hp_ref, row_ref, cole_ref, val_ref,
                  w_ref, b_ref, out_ref):
    bins = bins_ref[...]
    colcnt = col_ref[...]
    kk = lax.broadcasted_iota(jnp.int32, (_BW, 2 * _L), 0)
    vv = lax.broadcasted_iota(jnp.int32, (_BW, 2 * _L), 1)
    lo = ((kk % _L) == vv).astype(jnp.float32)
    hi = ((kk % _L) == (vv - _L)).astype(jnp.float32)
    is_lo = (kk < _L * _L).astype(jnp.float32)
    sel = lo * is_lo + hi * (1.0 - is_lo)
    counts = jnp.dot(bins, sel, preferred_element_type=jnp.float32)
    valcnt = counts[:, :_L]
    rowcnt = counts[:, _L:]

    vmask = (lax.broadcasted_iota(jnp.int32, (_VOCAB, 1), 0) != 0
             ).astype(jnp.float32)
    vtab = jnp.concatenate(
        [val_ref[...] * vmask, jnp.zeros((_L - _VOCAB, _NE), jnp.float32)],
        axis=0)
    num = (jnp.dot(valcnt, vtab, preferred_element_type=jnp.float32)
           + jnp.dot(rowcnt, row_ref[...], preferred_element_type=jnp.float32)
           + jnp.dot(colcnt, cole_ref[...],
                     preferred_element_type=jnp.float32))
    den = jnp.maximum(float(_J) - valcnt[:, 0:1], 1.0)
    h = num / den
    dn = (((1,), (1,)), ((), ()))
    out = lax.dot_general(h, w_ref[:, :_NE], dn,
                          preferred_element_type=jnp.float32)
    out = out + lax.dot_general(hp_ref[...], w_ref[:, _NE:], dn,
                                preferred_element_type=jnp.float32)
    out_ref[...] = out + b_ref[...]


@jax.jit
def kernel(x, h_parent, row_embed, col_embed, val_embed, head_w, head_b):
    x2 = x.reshape(_B * _J).astype(jnp.int32)
    bins, colcnt = _sc_scatter(x2)
    nd = head_w.shape[0]
    out = pl.pallas_call(
        _combine_body,
        out_shape=jax.ShapeDtypeStruct((_B, nd), jnp.float32),
    )(bins.reshape(_B, _BW), colcnt.reshape(_B, _L), h_parent, row_embed,
      col_embed, val_embed, head_w, head_b.reshape(1, -1))
    return out


# speedup vs baseline: 1.0676x; 1.0676x over previous
import jax
import jax.numpy as jnp
from jax import lax
from jax.experimental import pallas as pl
from jax.experimental.pallas import tpu as pltpu
from jax.experimental.pallas import tpu_sc as plsc

_B, _T, _H, _W = 1024, 8, 16, 16
_J = _T * _H * _W
_NE = 64
_VOCAB = 10
_NC, _NS, _L = 2, 16, 16
_NW = _NC * _NS
_SPW = _B // _NW
_CW = 64


def _counts_body(x_hbm, out_hbm, xall, hist, rmat, ostage):
    cid = lax.axis_index("c")
    sid = lax.axis_index("s")
    wid = sid * _NC + cid
    base = wid * _SPW
    pltpu.sync_copy(x_hbm.at[pl.ds(base * _J, _SPW * _J)], xall)
    lane = lax.iota(jnp.int32, _L)
    lane16 = lane * _L
    ones = jnp.ones((_L,), jnp.float32)
    zeros16 = jnp.zeros((_L,), jnp.float32)

    def sample_body(s, carry):
        for l in range(_L):
            hist[pl.ds(l * _L, _L)] = zeros16
            rmat[pl.ds(l * _L, _L)] = zeros16
        soff = s * _J

        @plsc.parallel_loop(0, _J // _L, carry=zeros16, unroll=8)
        def colacc(i, acc):
            xv = xall[pl.ds(soff + i * _L, _L)]
            maskf = jnp.minimum(xv, 1).astype(jnp.float32)
            plsc.addupdate_scatter(hist, [lane16 + xv], ones)
            plsc.addupdate_scatter(rmat, [lane16 + i % _H], maskf)
            return acc + maskf

        valcnt = hist[pl.ds(0, _L)]
        rowcnt = rmat[pl.ds(0, _L)]
        for l in range(1, _L):
            valcnt = valcnt + hist[pl.ds(l * _L, _L)]
            rowcnt = rowcnt + rmat[pl.ds(l * _L, _L)]
        ostage[s, pl.ds(0, _L)] = rowcnt
        ostage[s, pl.ds(_L, _L)] = colacc
        ostage[s, pl.ds(2 * _L, _L)] = valcnt
        ostage[s, pl.ds(3 * _L, _L)] = zeros16
        return carry
    lax.fori_loop(0, _SPW, sample_body, 0)
    pltpu.sync_copy(ostage, out_hbm.at[pl.ds(base, _SPW)])


def _sc_counts(x2):
    mesh = plsc.VectorSubcoreMesh(core_axis_name="c", subcore_axis_name="s",
                                  num_cores=_NC, num_subcores=_NS)
    fn = pl.kernel(
        _counts_body,
        out_type=jax.ShapeDtypeStruct((_B, _CW), jnp.float32),
        mesh=mesh,
        compiler_params=pltpu.CompilerParams(needs_layout_passes=False),
        scratch_types=[
            pltpu.VMEM((_SPW * _J,), jnp.int32),
            pltpu.VMEM((_L * _L,), jnp.float32),
            pltpu.VMEM((_L * _L,), jnp.float32),
            pltpu.VMEM((_SPW, _CW), jnp.float32),
        ],
    )
    return fn(x2)


def _combine_body(cnt_ref, hp_ref, row_ref, col_ref, val_ref, w_ref, b_ref,
                  out_ref):
    counts = cnt_ref[...]
    vmask = (lax.broadcasted_iota(jnp.int32, (_VOCAB, 1), 0) != 0
             ).astype(jnp.float32)
    table = jnp.concatenate(
        [row_ref[...], col_ref[...], val_ref[...] * vmask,
         jnp.zeros((_CW - 2 * _H - _VOCAB, _NE), jnp.float32)], axis=0)
    num = jnp.dot(counts, table, preferred_element_type=jnp.float32)
    cnt0 = counts[:, 2 * _L:2 * _L + 1]
    den = jnp.maximum(float(_J) - cnt0, 1.0)
    h = num / den
    dn = (((1,), (1,)), ((), ()))
    out = lax.dot_general(h, w_ref[:, :_NE], dn,
                          preferred_element_type=jnp.float32)
    out = out + lax.dot_general(hp_ref[...], w_ref[:, _NE:], dn,
                                preferred_element_type=jnp.float32)
    out_ref[...] = out + b_ref[...]


@jax.jit
def kernel(x, h_parent, row_embed, col_embed, val_embed, head_w, head_b):
    x2 = x.reshape(_B * _J).astype(jnp.int32)
    counts = _sc_counts(x2)
    nd = head_w.shape[0]
    out = pl.pallas_call(
        _combine_body,
        out_shape=jax.ShapeDtypeStruct((_B, nd), jnp.float32),
    )(counts, h_parent, row_embed, col_embed, val_embed, head_w,
      head_b.reshape(1, -1))
    return out
